# baseline (speedup 1.0000x reference)
import functools
import math

import jax
import jax.numpy as jnp
from jax import lax
from jax.experimental import pallas as pl
from jax.experimental.pallas import tpu as pltpu
from jax.experimental.pallas import tpu_sc as plsc

B = 4
S = 2048
D_MODEL = 1024
RANK = 64
N_COMPRESS = 64
N_KNOWLEDGE = 8192
K_KNOW = 8
TOPK_C = 16

TS = 128
NEG = -1e30
BIGIDX = 2**30

NW = 32
TOK = B * S
TPW = TOK // NW
GB = 4
NBATCH = TPW // GB
ROWS_PER_BATCH = GB * K_KNOW


def _mix_kernel(w_ref, idx_ref, cn_ref, out_ref):
    iota3 = jax.lax.broadcasted_iota(jnp.int32, (B, TOPK_C, N_COMPRESS), 2)
    oh = (iota3 == idx_ref[...][..., None]).astype(jnp.bfloat16)
    cn_bf = cn_ref[...].astype(jnp.bfloat16)
    w_bf = w_ref[...].astype(jnp.bfloat16)
    for b in range(B):
        rows = jnp.dot(oh[b], cn_bf, preferred_element_type=jnp.float32)
        shb = jnp.dot(w_bf[b:b + 1], rows.astype(jnp.bfloat16),
                      preferred_element_type=jnp.float32)
        out_ref[b:b + 1, :] = shb


NCLS = 512
NSL = N_KNOWLEDGE // NCLS


def _fold_top3(s, excl_mask=None):
    shape = (s.shape[0], NCLS)
    m1 = jnp.full(shape, NEG, jnp.float32)
    m2 = jnp.full(shape, NEG, jnp.float32)
    m3 = jnp.full(shape, NEG, jnp.float32)
    a1 = jnp.zeros(shape, jnp.int32)
    a2 = jnp.zeros(shape, jnp.int32)
    a3 = jnp.zeros(shape, jnp.int32)
    for t in range(NSL):
        x = s[:, t * NCLS:(t + 1) * NCLS]
        if excl_mask is not None:
            x = jnp.where(excl_mask[:, t * NCLS:(t + 1) * NCLS], NEG, x)
        c1 = x > m1
        c2 = x > m2
        c3 = x > m3
        m3 = jnp.where(c2, m2, jnp.where(c3, x, m3))
        a3 = jnp.where(c2, a2, jnp.where(c3, t, a3))
        m2 = jnp.where(c1, m1, jnp.where(c2, x, m2))
        a2 = jnp.where(c1, a1, jnp.where(c2, t, a2))
        m1 = jnp.where(c1, x, m1)
        a1 = jnp.where(c1, t, a1)
    return m1, a1, m2, a2, m3, a3


def _topk_kernel(x_ref, sc_ref, k_ref, idx_ref, w_ref):
    x = x_ref[0].astype(jnp.bfloat16)
    shared_c = sc_ref[0].astype(jnp.bfloat16)
    q = jnp.dot(x, shared_c, preferred_element_type=jnp.float32)
    s = jax.lax.dot_general(
        q.astype(jnp.bfloat16), k_ref[...].astype(jnp.bfloat16),
        (((1,), (1,)), ((), ())),
        preferred_element_type=jnp.float32) * (1.0 / math.sqrt(RANK))

    iota_c = jax.lax.broadcasted_iota(jnp.int32, (TS, NCLS), 1)
    iota_g = jax.lax.broadcasted_iota(jnp.int32, (TS, N_KNOWLEDGE), 1)

    p1, b1, p2, b2, p3, b3 = _fold_top3(s)
    vals = []
    idxs = []
    cnt = jnp.zeros((TS, NCLS), jnp.int32)
    for j in range(K_KNOW):
        m = jnp.max(p1, axis=1, keepdims=True)
        gcand = jnp.where(p1 == m, b1 * NCLS + iota_c, BIGIDX)
        a = jnp.min(gcand, axis=1, keepdims=True)
        vals.append(m)
        idxs.append(a)
        g = jnp.bitwise_and(a, NCLS - 1)
        hit = iota_c == g
        cnt = cnt + hit.astype(jnp.int32)
        p1 = jnp.where(hit, p2, p1)
        b1 = jnp.where(hit, b2, b1)
        p2 = jnp.where(hit, p3, p2)
        b2 = jnp.where(hit, b3, b2)
        p3 = jnp.where(hit, NEG, p3)

    v8 = jnp.concatenate(vals, axis=1)
    i8 = jnp.concatenate(idxs, axis=1)
    e8 = jnp.exp(v8 - v8[:, 0:1])
    w8 = e8 / jnp.sum(e8, axis=1, keepdims=True)
    idx_ref[0] = i8
    w_ref[0] = w8

    bad = jnp.any(cnt >= 3)

    @pl.when(bad)
    def _slow_exact():
        ss = s
        nvals = []
        nidxs = []
        for _ in range(K_KNOW):
            nm = jnp.max(ss, axis=1, keepdims=True)
            ncand = jnp.where(ss == nm, iota_g, BIGIDX)
            na = jnp.min(ncand, axis=1, keepdims=True)
            nvals.append(nm)
            nidxs.append(na)
            ss = jnp.where(iota_g == na, NEG, ss)
        nv8 = jnp.concatenate(nvals, axis=1)
        ni8 = jnp.concatenate(nidxs, axis=1)
        ne8 = jnp.exp(nv8 - nv8[:, 0:1])
        nw8 = ne8 / jnp.sum(ne8, axis=1, keepdims=True)
        idx_ref[0] = ni8
        w_ref[0] = nw8


def _sc_combine(idx_hbm, w_hbm, v_hbm, out_hbm,
                idx_v, w_v, rows_v, outb_v, rsem, osem):
    wid = lax.axis_index("s") * 2 + lax.axis_index("c")
    base = wid * TPW
    pltpu.sync_copy(idx_hbm.at[wid], idx_v)
    pltpu.sync_copy(w_hbm.at[wid], w_v)

    pltpu.async_copy(v_hbm.at[idx_v.at[0]], rows_v.at[0], rsem.at[0])
    pltpu.async_copy(v_hbm.at[idx_v.at[1]], rows_v.at[1], rsem.at[1])

    def batch_pair(gg, carry):
        for par in range(2):
            g = gg * 2 + par
            pltpu.make_async_copy(
                v_hbm.at[idx_v.at[g]], rows_v.at[par], rsem.at[par]).wait()
            @pl.when(gg > 0)
            def _():
                pltpu.make_async_copy(
                    outb_v.at[par],
                    out_hbm.at[pl.ds(base, GB)],
                    osem.at[par]).wait()

            wpos0 = g * (GB * K_KNOW)
            w_chunks = [w_v[pl.ds(wpos0, 16)], w_v[pl.ds(wpos0 + 16, 16)]]
            ws = []
            for t in range(GB):
                wst = []
                for j in range(K_KNOW):
                    p = t * K_KNOW + j
                    lane = jnp.full((16,), p % 16, jnp.int32)
                    wst.append(w_chunks[p // 16].at[lane]
                               .get(mode="promise_in_bounds"))
                ws.append(wst)

            def cbody(c, _):
                sl = pl.ds(c * 16, 16)
                for t in range(GB):
                    a = rows_v[par, t * K_KNOW + 0, sl] * ws[t][0]
                    for j in range(1, K_KNOW):
                        a = a + rows_v[par, t * K_KNOW + j, sl] * ws[t][j]
                    outb_v[par, t, sl] = a
                return 0

            jax.lax.fori_loop(0, D_MODEL // 16, cbody, 0)

            pltpu.async_copy(
                outb_v.at[par],
                out_hbm.at[pl.ds(base + g * GB, GB)],
                osem.at[par])
            @pl.when(gg < NBATCH // 2 - 1)
            def _():
                pltpu.async_copy(
                    v_hbm.at[idx_v.at[g + 2]], rows_v.at[par], rsem.at[par])
        return carry

    jax.lax.fori_loop(0, NBATCH // 2, batch_pair, 0)

    for par in range(2):
        pltpu.make_async_copy(
            outb_v.at[par], out_hbm.at[pl.ds(base, GB)], osem.at[par]).wait()


def kernel(x, memory_topk_w, memory_topk_idx, compress_neurons, knowledge_K, knowledge_V):
    cn2 = compress_neurons.reshape(N_COMPRESS, D_MODEL * RANK)
    shared_flat = pl.pallas_call(
        _mix_kernel,
        grid=(16,),
        in_specs=[
            pl.BlockSpec((B, TOPK_C), lambda i: (0, 0)),
            pl.BlockSpec((B, TOPK_C), lambda i: (0, 0)),
            pl.BlockSpec((N_COMPRESS, D_MODEL * RANK // 16), lambda i: (0, i)),
        ],
        out_specs=pl.BlockSpec((B, D_MODEL * RANK // 16), lambda i: (0, i)),
        out_shape=jax.ShapeDtypeStruct((B, D_MODEL * RANK), jnp.float32),
    )(memory_topk_w, memory_topk_idx, cn2)
    shared_compress = shared_flat.reshape(B, D_MODEL, RANK)

    topk_idx, weights = pl.pallas_call(
        _topk_kernel,
        grid=(B, S // TS),
        in_specs=[
            pl.BlockSpec((1, TS, D_MODEL), lambda b, s: (b, s, 0)),
            pl.BlockSpec((1, D_MODEL, RANK), lambda b, s: (b, 0, 0)),
            pl.BlockSpec((N_KNOWLEDGE, RANK), lambda b, s: (0, 0)),
        ],
        out_specs=[
            pl.BlockSpec((1, TS, K_KNOW), lambda b, s: (b, s, 0)),
            pl.BlockSpec((1, TS, K_KNOW), lambda b, s: (b, s, 0)),
        ],
        out_shape=[
            jax.ShapeDtypeStruct((B, S, K_KNOW), jnp.int32),
            jax.ShapeDtypeStruct((B, S, K_KNOW), jnp.float32),
        ],
    )(x, shared_compress, knowledge_K)

    idx_w = topk_idx.reshape(NW, NBATCH, ROWS_PER_BATCH)
    w_w = weights.reshape(NW, TPW * K_KNOW)

    mesh = plsc.VectorSubcoreMesh(core_axis_name="c", subcore_axis_name="s")
    out_flat = pl.kernel(
        _sc_combine,
        mesh=mesh,
        out_type=jax.ShapeDtypeStruct((TOK, D_MODEL), jnp.float32),
        scratch_types=[
            pltpu.VMEM((NBATCH, ROWS_PER_BATCH), jnp.int32),
            pltpu.VMEM((TPW * K_KNOW,), jnp.float32),
            pltpu.VMEM((2, ROWS_PER_BATCH, D_MODEL), jnp.float32),
            pltpu.VMEM((2, GB, D_MODEL), jnp.float32),
            pltpu.SemaphoreType.DMA((2,)),
            pltpu.SemaphoreType.DMA((2,)),
        ],
    )(idx_w, w_w, knowledge_V)
    output = out_flat.reshape(B, S, D_MODEL)
    return (output, topk_idx, weights)


# speedup vs baseline: 1.0977x; 1.0977x over previous
import functools
import math

import jax
import jax.numpy as jnp
from jax import lax
from jax.experimental import pallas as pl
from jax.experimental.pallas import tpu as pltpu
from jax.experimental.pallas import tpu_sc as plsc

B = 4
S = 2048
D_MODEL = 1024
RANK = 64
N_COMPRESS = 64
N_KNOWLEDGE = 8192
K_KNOW = 8
TOPK_C = 16

TS = 128
NEG = -1e30
BIGIDX = 2**30

NW = 32
TOK = B * S
TPW = TOK // NW
GB = 4
NBATCH = TPW // GB
ROWS_PER_BATCH = GB * K_KNOW


def _mix_kernel(w_ref, idx_ref, cn_ref, out_ref):
    iota3 = jax.lax.broadcasted_iota(jnp.int32, (B, TOPK_C, N_COMPRESS), 2)
    oh = (iota3 == idx_ref[...][..., None]).astype(jnp.bfloat16)
    cn_bf = cn_ref[...].astype(jnp.bfloat16)
    w_bf = w_ref[...].astype(jnp.bfloat16)
    for b in range(B):
        rows = jnp.dot(oh[b], cn_bf, preferred_element_type=jnp.float32)
        shb = jnp.dot(w_bf[b:b + 1], rows.astype(jnp.bfloat16),
                      preferred_element_type=jnp.float32)
        out_ref[b:b + 1, :] = shb


NCLS = 512
NSL = N_KNOWLEDGE // NCLS


def _fold_top3(s, excl_mask=None):
    shape = (s.shape[0], NCLS)
    m1 = jnp.full(shape, NEG, jnp.float32)
    m2 = jnp.full(shape, NEG, jnp.float32)
    m3 = jnp.full(shape, NEG, jnp.float32)
    a1 = jnp.zeros(shape, jnp.int32)
    a2 = jnp.zeros(shape, jnp.int32)
    a3 = jnp.zeros(shape, jnp.int32)
    for t in range(NSL):
        x = s[:, t * NCLS:(t + 1) * NCLS]
        if excl_mask is not None:
            x = jnp.where(excl_mask[:, t * NCLS:(t + 1) * NCLS], NEG, x)
        c1 = x > m1
        c2 = x > m2
        c3 = x > m3
        m3 = jnp.where(c2, m2, jnp.where(c3, x, m3))
        a3 = jnp.where(c2, a2, jnp.where(c3, t, a3))
        m2 = jnp.where(c1, m1, jnp.where(c2, x, m2))
        a2 = jnp.where(c1, a1, jnp.where(c2, t, a2))
        m1 = jnp.where(c1, x, m1)
        a1 = jnp.where(c1, t, a1)
    return m1, a1, m2, a2, m3, a3


def _topk_kernel(x_ref, sc_ref, k_ref, idx_ref, w_ref):
    x = x_ref[0].astype(jnp.bfloat16)
    shared_c = sc_ref[0].astype(jnp.bfloat16)
    q = jnp.dot(x, shared_c, preferred_element_type=jnp.float32)
    s = jax.lax.dot_general(
        q.astype(jnp.bfloat16), k_ref[...].astype(jnp.bfloat16),
        (((1,), (1,)), ((), ())),
        preferred_element_type=jnp.float32) * (1.0 / math.sqrt(RANK))

    iota_c = jax.lax.broadcasted_iota(jnp.int32, (TS, NCLS), 1)
    iota_g = jax.lax.broadcasted_iota(jnp.int32, (TS, N_KNOWLEDGE), 1)

    p1, b1, p2, b2, p3, b3 = _fold_top3(s)
    vals = []
    idxs = []
    cnt = jnp.zeros((TS, NCLS), jnp.int32)
    for j in range(K_KNOW):
        m = jnp.max(p1, axis=1, keepdims=True)
        gcand = jnp.where(p1 == m, b1 * NCLS + iota_c, BIGIDX)
        a = jnp.min(gcand, axis=1, keepdims=True)
        vals.append(m)
        idxs.append(a)
        g = jnp.bitwise_and(a, NCLS - 1)
        hit = iota_c == g
        cnt = cnt + hit.astype(jnp.int32)
        p1 = jnp.where(hit, p2, p1)
        b1 = jnp.where(hit, b2, b1)
        p2 = jnp.where(hit, p3, p2)
        b2 = jnp.where(hit, b3, b2)
        p3 = jnp.where(hit, NEG, p3)

    v8 = jnp.concatenate(vals, axis=1)
    i8 = jnp.concatenate(idxs, axis=1)
    e8 = jnp.exp(v8 - v8[:, 0:1])
    w8 = e8 / jnp.sum(e8, axis=1, keepdims=True)
    idx_ref[0] = i8
    w_ref[0] = w8

    bad = jnp.any(cnt >= 3)

    @pl.when(bad)
    def _slow_exact():
        ss = s
        nvals = []
        nidxs = []
        for _ in range(K_KNOW):
            nm = jnp.max(ss, axis=1, keepdims=True)
            ncand = jnp.where(ss == nm, iota_g, BIGIDX)
            na = jnp.min(ncand, axis=1, keepdims=True)
            nvals.append(nm)
            nidxs.append(na)
            ss = jnp.where(iota_g == na, NEG, ss)
        nv8 = jnp.concatenate(nvals, axis=1)
        ni8 = jnp.concatenate(nidxs, axis=1)
        ne8 = jnp.exp(nv8 - nv8[:, 0:1])
        nw8 = ne8 / jnp.sum(ne8, axis=1, keepdims=True)
        idx_ref[0] = ni8
        w_ref[0] = nw8


def _make_sc_combine(tpw, nbatch):
    def _sc_combine(idx_hbm, w_hbm, v_hbm, out_hbm,
                    idx_v, w_v, rows_v, outb_v, rsem, osem):
        wid = lax.axis_index("s") * 2 + lax.axis_index("c")
        base = wid * tpw
        pltpu.sync_copy(idx_hbm.at[wid], idx_v)
        pltpu.sync_copy(w_hbm.at[wid], w_v)

        pltpu.async_copy(v_hbm.at[idx_v.at[0]], rows_v.at[0], rsem.at[0])
        pltpu.async_copy(v_hbm.at[idx_v.at[1]], rows_v.at[1], rsem.at[1])

        def batch_pair(gg, carry):
            for par in range(2):
                g = gg * 2 + par
                pltpu.make_async_copy(
                    v_hbm.at[idx_v.at[g]], rows_v.at[par], rsem.at[par]).wait()
                @pl.when(gg > 0)
                def _():
                    pltpu.make_async_copy(
                        outb_v.at[par],
                        out_hbm.at[pl.ds(base, GB)],
                        osem.at[par]).wait()

                wpos0 = g * (GB * K_KNOW)
                w_chunks = [w_v[pl.ds(wpos0, 16)], w_v[pl.ds(wpos0 + 16, 16)]]
                ws = []
                for t in range(GB):
                    wst = []
                    for j in range(K_KNOW):
                        p = t * K_KNOW + j
                        lane = jnp.full((16,), p % 16, jnp.int32)
                        wst.append(w_chunks[p // 16].at[lane]
                                   .get(mode="promise_in_bounds"))
                    ws.append(wst)

                def cbody(c, _):
                    sl = pl.ds(c * 16, 16)
                    for t in range(GB):
                        a = rows_v[par, t * K_KNOW + 0, sl] * ws[t][0]
                        for j in range(1, K_KNOW):
                            a = a + rows_v[par, t * K_KNOW + j, sl] * ws[t][j]
                        outb_v[par, t, sl] = a
                    return 0

                jax.lax.fori_loop(0, D_MODEL // 16, cbody, 0)

                pltpu.async_copy(
                    outb_v.at[par],
                    out_hbm.at[pl.ds(base + g * GB, GB)],
                    osem.at[par])
                @pl.when(gg < nbatch // 2 - 1)
                def _():
                    pltpu.async_copy(
                        v_hbm.at[idx_v.at[g + 2]], rows_v.at[par], rsem.at[par])
            return carry

        jax.lax.fori_loop(0, nbatch // 2, batch_pair, 0)

        for par in range(2):
            pltpu.make_async_copy(
                outb_v.at[par], out_hbm.at[pl.ds(base, GB)], osem.at[par]).wait()

    return _sc_combine


NCH = 2
S_CH = S // NCH


def kernel(x, memory_topk_w, memory_topk_idx, compress_neurons, knowledge_K, knowledge_V):
    cn2 = compress_neurons.reshape(N_COMPRESS, D_MODEL * RANK)
    shared_flat = pl.pallas_call(
        _mix_kernel,
        grid=(16,),
        in_specs=[
            pl.BlockSpec((B, TOPK_C), lambda i: (0, 0)),
            pl.BlockSpec((B, TOPK_C), lambda i: (0, 0)),
            pl.BlockSpec((N_COMPRESS, D_MODEL * RANK // 16), lambda i: (0, i)),
        ],
        out_specs=pl.BlockSpec((B, D_MODEL * RANK // 16), lambda i: (0, i)),
        out_shape=jax.ShapeDtypeStruct((B, D_MODEL * RANK), jnp.float32),
    )(memory_topk_w, memory_topk_idx, cn2)
    shared_compress = shared_flat.reshape(B, D_MODEL, RANK)

    tok_ch = B * S_CH
    tpw = tok_ch // NW
    nbatch = tpw // GB
    mesh = plsc.VectorSubcoreMesh(core_axis_name="c", subcore_axis_name="s")
    sc_body = _make_sc_combine(tpw, nbatch)

    out_chunks, idx_chunks, w_chunks = [], [], []
    for c in range(NCH):
        xc = x[:, c * S_CH:(c + 1) * S_CH]
        topk_idx_c, weights_c = pl.pallas_call(
            _topk_kernel,
            grid=(B, S_CH // TS),
            in_specs=[
                pl.BlockSpec((1, TS, D_MODEL), lambda b, s: (b, s, 0)),
                pl.BlockSpec((1, D_MODEL, RANK), lambda b, s: (b, 0, 0)),
                pl.BlockSpec((N_KNOWLEDGE, RANK), lambda b, s: (0, 0)),
            ],
            out_specs=[
                pl.BlockSpec((1, TS, K_KNOW), lambda b, s: (b, s, 0)),
                pl.BlockSpec((1, TS, K_KNOW), lambda b, s: (b, s, 0)),
            ],
            out_shape=[
                jax.ShapeDtypeStruct((B, S_CH, K_KNOW), jnp.int32),
                jax.ShapeDtypeStruct((B, S_CH, K_KNOW), jnp.float32),
            ],
        )(xc, shared_compress, knowledge_K)

        idx_w = topk_idx_c.reshape(NW, nbatch, ROWS_PER_BATCH)
        w_w = weights_c.reshape(NW, tpw * K_KNOW)
        out_flat = pl.kernel(
            sc_body,
            mesh=mesh,
            out_type=jax.ShapeDtypeStruct((tok_ch, D_MODEL), jnp.float32),
            scratch_types=[
                pltpu.VMEM((nbatch, ROWS_PER_BATCH), jnp.int32),
                pltpu.VMEM((tpw * K_KNOW,), jnp.float32),
                pltpu.VMEM((2, ROWS_PER_BATCH, D_MODEL), jnp.float32),
                pltpu.VMEM((2, GB, D_MODEL), jnp.float32),
                pltpu.SemaphoreType.DMA((2,)),
                pltpu.SemaphoreType.DMA((2,)),
            ],
        )(idx_w, w_w, knowledge_V)
        out_chunks.append(out_flat.reshape(B, S_CH, D_MODEL))
        idx_chunks.append(topk_idx_c)
        w_chunks.append(weights_c)

    output = jnp.concatenate(out_chunks, axis=1)
    topk_idx = jnp.concatenate(idx_chunks, axis=1)
    weights = jnp.concatenate(w_chunks, axis=1)
    return (output, topk_idx, weights)


# speedup vs baseline: 1.1819x; 1.0766x over previous
import functools
import math

import jax
import jax.numpy as jnp
from jax import lax
from jax.experimental import pallas as pl
from jax.experimental.pallas import tpu as pltpu
from jax.experimental.pallas import tpu_sc as plsc

B = 4
S = 2048
D_MODEL = 1024
RANK = 64
N_COMPRESS = 64
N_KNOWLEDGE = 8192
K_KNOW = 8
TOPK_C = 16

TS = 128
NEG = -1e30
BIGIDX = 2**30

NW = 32
TOK = B * S
TPW = TOK // NW
GB = 4
NBATCH = TPW // GB
ROWS_PER_BATCH = GB * K_KNOW


def _mix_kernel(w_ref, idx_ref, cn_ref, out_ref):
    iota3 = jax.lax.broadcasted_iota(jnp.int32, (B, TOPK_C, N_COMPRESS), 2)
    oh = (iota3 == idx_ref[...][..., None]).astype(jnp.bfloat16)
    cn_bf = cn_ref[...].astype(jnp.bfloat16)
    w_bf = w_ref[...].astype(jnp.bfloat16)
    for b in range(B):
        rows = jnp.dot(oh[b], cn_bf, preferred_element_type=jnp.float32)
        shb = jnp.dot(w_bf[b:b + 1], rows.astype(jnp.bfloat16),
                      preferred_element_type=jnp.float32)
        out_ref[b:b + 1, :] = shb


NCLS = 512
NSL = N_KNOWLEDGE // NCLS


def _fold_top3(s, excl_mask=None):
    shape = (s.shape[0], NCLS)
    m1 = jnp.full(shape, NEG, jnp.float32)
    m2 = jnp.full(shape, NEG, jnp.float32)
    m3 = jnp.full(shape, NEG, jnp.float32)
    a1 = jnp.zeros(shape, jnp.int32)
    a2 = jnp.zeros(shape, jnp.int32)
    a3 = jnp.zeros(shape, jnp.int32)
    for t in range(NSL):
        x = s[:, t * NCLS:(t + 1) * NCLS]
        if excl_mask is not None:
            x = jnp.where(excl_mask[:, t * NCLS:(t + 1) * NCLS], NEG, x)
        c1 = x > m1
        c2 = x > m2
        c3 = x > m3
        m3 = jnp.where(c2, m2, jnp.where(c3, x, m3))
        a3 = jnp.where(c2, a2, jnp.where(c3, t, a3))
        m2 = jnp.where(c1, m1, jnp.where(c2, x, m2))
        a2 = jnp.where(c1, a1, jnp.where(c2, t, a2))
        m1 = jnp.where(c1, x, m1)
        a1 = jnp.where(c1, t, a1)
    return m1, a1, m2, a2, m3, a3


def _topk_kernel(x_ref, sc_ref, k_ref, idx_ref, w_ref):
    x = x_ref[0].astype(jnp.bfloat16)
    shared_c = sc_ref[0].astype(jnp.bfloat16)
    q = jnp.dot(x, shared_c, preferred_element_type=jnp.float32)
    s = jax.lax.dot_general(
        q.astype(jnp.bfloat16), k_ref[...].astype(jnp.bfloat16),
        (((1,), (1,)), ((), ())),
        preferred_element_type=jnp.float32) * (1.0 / math.sqrt(RANK))

    iota_c = jax.lax.broadcasted_iota(jnp.int32, (TS, NCLS), 1)
    iota_g = jax.lax.broadcasted_iota(jnp.int32, (TS, N_KNOWLEDGE), 1)

    p1, b1, p2, b2, p3, b3 = _fold_top3(s)
    vals = []
    idxs = []
    cnt = jnp.zeros((TS, NCLS), jnp.int32)
    for j in range(K_KNOW):
        m = jnp.max(p1, axis=1, keepdims=True)
        gcand = jnp.where(p1 == m, b1 * NCLS + iota_c, BIGIDX)
        a = jnp.min(gcand, axis=1, keepdims=True)
        vals.append(m)
        idxs.append(a)
        g = jnp.bitwise_and(a, NCLS - 1)
        hit = iota_c == g
        cnt = cnt + hit.astype(jnp.int32)
        p1 = jnp.where(hit, p2, p1)
        b1 = jnp.where(hit, b2, b1)
        p2 = jnp.where(hit, p3, p2)
        b2 = jnp.where(hit, b3, b2)
        p3 = jnp.where(hit, NEG, p3)

    v8 = jnp.concatenate(vals, axis=1)
    i8 = jnp.concatenate(idxs, axis=1)
    e8 = jnp.exp(v8 - v8[:, 0:1])
    w8 = e8 / jnp.sum(e8, axis=1, keepdims=True)
    idx_ref[0] = i8
    w_ref[0] = w8

    bad = jnp.any(cnt >= 3)

    @pl.when(bad)
    def _slow_exact():
        ss = s
        nvals = []
        nidxs = []
        for _ in range(K_KNOW):
            nm = jnp.max(ss, axis=1, keepdims=True)
            ncand = jnp.where(ss == nm, iota_g, BIGIDX)
            na = jnp.min(ncand, axis=1, keepdims=True)
            nvals.append(nm)
            nidxs.append(na)
            ss = jnp.where(iota_g == na, NEG, ss)
        nv8 = jnp.concatenate(nvals, axis=1)
        ni8 = jnp.concatenate(nidxs, axis=1)
        ne8 = jnp.exp(nv8 - nv8[:, 0:1])
        nw8 = ne8 / jnp.sum(ne8, axis=1, keepdims=True)
        idx_ref[0] = ni8
        w_ref[0] = nw8


def _make_sc_combine(tpw, nbatch):
    def _sc_combine(idx_hbm, w_hbm, v_hbm, out_hbm,
                    idx_v, w_v, rows_v, outb_v, rsem, osem):
        wid = lax.axis_index("s") * 2 + lax.axis_index("c")
        base = wid * tpw
        pltpu.sync_copy(idx_hbm.at[wid], idx_v)
        pltpu.sync_copy(w_hbm.at[wid], w_v)

        pltpu.async_copy(v_hbm.at[idx_v.at[0]], rows_v.at[0], rsem.at[0])
        pltpu.async_copy(v_hbm.at[idx_v.at[1]], rows_v.at[1], rsem.at[1])

        def batch_pair(gg, carry):
            for par in range(2):
                g = gg * 2 + par
                pltpu.make_async_copy(
                    v_hbm.at[idx_v.at[g]], rows_v.at[par], rsem.at[par]).wait()
                @pl.when(gg > 0)
                def _():
                    pltpu.make_async_copy(
                        outb_v.at[par],
                        out_hbm.at[pl.ds(base, GB)],
                        osem.at[par]).wait()

                wpos0 = g * (GB * K_KNOW)
                w_chunks = [w_v[pl.ds(wpos0, 16)], w_v[pl.ds(wpos0 + 16, 16)]]
                ws = []
                for t in range(GB):
                    wst = []
                    for j in range(K_KNOW):
                        p = t * K_KNOW + j
                        lane = jnp.full((16,), p % 16, jnp.int32)
                        wst.append(w_chunks[p // 16].at[lane]
                                   .get(mode="promise_in_bounds"))
                    ws.append(wst)

                def cbody(c, _):
                    sl = pl.ds(c * 16, 16)
                    for t in range(GB):
                        a = rows_v[par, t * K_KNOW + 0, sl] * ws[t][0]
                        for j in range(1, K_KNOW):
                            a = a + rows_v[par, t * K_KNOW + j, sl] * ws[t][j]
                        outb_v[par, t, sl] = a
                    return 0

                jax.lax.fori_loop(0, D_MODEL // 16, cbody, 0)

                pltpu.async_copy(
                    outb_v.at[par],
                    out_hbm.at[pl.ds(base + g * GB, GB)],
                    osem.at[par])
                @pl.when(gg < nbatch // 2 - 1)
                def _():
                    pltpu.async_copy(
                        v_hbm.at[idx_v.at[g + 2]], rows_v.at[par], rsem.at[par])
            return carry

        jax.lax.fori_loop(0, nbatch // 2, batch_pair, 0)

        for par in range(2):
            pltpu.make_async_copy(
                outb_v.at[par], out_hbm.at[pl.ds(base, GB)], osem.at[par]).wait()

    return _sc_combine


NCH = 4
S_CH = S // NCH


def kernel(x, memory_topk_w, memory_topk_idx, compress_neurons, knowledge_K, knowledge_V):
    cn2 = compress_neurons.reshape(N_COMPRESS, D_MODEL * RANK)
    shared_flat = pl.pallas_call(
        _mix_kernel,
        grid=(16,),
        in_specs=[
            pl.BlockSpec((B, TOPK_C), lambda i: (0, 0)),
            pl.BlockSpec((B, TOPK_C), lambda i: (0, 0)),
            pl.BlockSpec((N_COMPRESS, D_MODEL * RANK // 16), lambda i: (0, i)),
        ],
        out_specs=pl.BlockSpec((B, D_MODEL * RANK // 16), lambda i: (0, i)),
        out_shape=jax.ShapeDtypeStruct((B, D_MODEL * RANK), jnp.float32),
    )(memory_topk_w, memory_topk_idx, cn2)
    shared_compress = shared_flat.reshape(B, D_MODEL, RANK)

    tok_ch = B * S_CH
    tpw = tok_ch // NW
    nbatch = tpw // GB
    mesh = plsc.VectorSubcoreMesh(core_axis_name="c", subcore_axis_name="s")
    sc_body = _make_sc_combine(tpw, nbatch)

    out_chunks, idx_chunks, w_chunks = [], [], []
    for c in range(NCH):
        xc = x[:, c * S_CH:(c + 1) * S_CH]
        topk_idx_c, weights_c = pl.pallas_call(
            _topk_kernel,
            grid=(B, S_CH // TS),
            in_specs=[
                pl.BlockSpec((1, TS, D_MODEL), lambda b, s: (b, s, 0)),
                pl.BlockSpec((1, D_MODEL, RANK), lambda b, s: (b, 0, 0)),
                pl.BlockSpec((N_KNOWLEDGE, RANK), lambda b, s: (0, 0)),
            ],
            out_specs=[
                pl.BlockSpec((1, TS, K_KNOW), lambda b, s: (b, s, 0)),
                pl.BlockSpec((1, TS, K_KNOW), lambda b, s: (b, s, 0)),
            ],
            out_shape=[
                jax.ShapeDtypeStruct((B, S_CH, K_KNOW), jnp.int32),
                jax.ShapeDtypeStruct((B, S_CH, K_KNOW), jnp.float32),
            ],
        )(xc, shared_compress, knowledge_K)

        idx_w = topk_idx_c.reshape(NW, nbatch, ROWS_PER_BATCH)
        w_w = weights_c.reshape(NW, tpw * K_KNOW)
        out_flat = pl.kernel(
            sc_body,
            mesh=mesh,
            out_type=jax.ShapeDtypeStruct((tok_ch, D_MODEL), jnp.float32),
            scratch_types=[
                pltpu.VMEM((nbatch, ROWS_PER_BATCH), jnp.int32),
                pltpu.VMEM((tpw * K_KNOW,), jnp.float32),
                pltpu.VMEM((2, ROWS_PER_BATCH, D_MODEL), jnp.float32),
                pltpu.VMEM((2, GB, D_MODEL), jnp.float32),
                pltpu.SemaphoreType.DMA((2,)),
                pltpu.SemaphoreType.DMA((2,)),
            ],
        )(idx_w, w_w, knowledge_V)
        out_chunks.append(out_flat.reshape(B, S_CH, D_MODEL))
        idx_chunks.append(topk_idx_c)
        w_chunks.append(weights_c)

    output = jnp.concatenate(out_chunks, axis=1)
    topk_idx = jnp.concatenate(idx_chunks, axis=1)
    weights = jnp.concatenate(w_chunks, axis=1)
    return (output, topk_idx, weights)


# speedup vs baseline: 1.1870x; 1.0044x over previous
import functools
import math

import jax
import jax.numpy as jnp
from jax import lax
from jax.experimental import pallas as pl
from jax.experimental.pallas import tpu as pltpu
from jax.experimental.pallas import tpu_sc as plsc

B = 4
S = 2048
D_MODEL = 1024
RANK = 64
N_COMPRESS = 64
N_KNOWLEDGE = 8192
K_KNOW = 8
TOPK_C = 16

TS = 128
NEG = -1e30
BIGIDX = 2**30

NW = 32
TOK = B * S
TPW = TOK // NW
GB = 4
NBATCH = TPW // GB
ROWS_PER_BATCH = GB * K_KNOW


def _mix_kernel(w_ref, idx_ref, cn_ref, out_ref):
    iota3 = jax.lax.broadcasted_iota(jnp.int32, (B, TOPK_C, N_COMPRESS), 2)
    oh = (iota3 == idx_ref[...][..., None]).astype(jnp.bfloat16)
    cn_bf = cn_ref[...].astype(jnp.bfloat16)
    w_bf = w_ref[...].astype(jnp.bfloat16)
    for b in range(B):
        rows = jnp.dot(oh[b], cn_bf, preferred_element_type=jnp.float32)
        shb = jnp.dot(w_bf[b:b + 1], rows.astype(jnp.bfloat16),
                      preferred_element_type=jnp.float32)
        out_ref[b:b + 1, :] = shb


NCLS = 512
NSL = N_KNOWLEDGE // NCLS


def _fold_top3(s, excl_mask=None):
    shape = (s.shape[0], NCLS)
    m1 = jnp.full(shape, NEG, jnp.float32)
    m2 = jnp.full(shape, NEG, jnp.float32)
    m3 = jnp.full(shape, NEG, jnp.float32)
    a1 = jnp.zeros(shape, jnp.int32)
    a2 = jnp.zeros(shape, jnp.int32)
    a3 = jnp.zeros(shape, jnp.int32)
    for t in range(NSL):
        x = s[:, t * NCLS:(t + 1) * NCLS]
        if excl_mask is not None:
            x = jnp.where(excl_mask[:, t * NCLS:(t + 1) * NCLS], NEG, x)
        c1 = x > m1
        c2 = x > m2
        c3 = x > m3
        m3 = jnp.where(c2, m2, jnp.where(c3, x, m3))
        a3 = jnp.where(c2, a2, jnp.where(c3, t, a3))
        m2 = jnp.where(c1, m1, jnp.where(c2, x, m2))
        a2 = jnp.where(c1, a1, jnp.where(c2, t, a2))
        m1 = jnp.where(c1, x, m1)
        a1 = jnp.where(c1, t, a1)
    return m1, a1, m2, a2, m3, a3


def _topk_kernel(x_ref, sc_ref, k_ref, idx_ref, w_ref):
    x = x_ref[0].astype(jnp.bfloat16)
    shared_c = sc_ref[0].astype(jnp.bfloat16)
    q = jnp.dot(x, shared_c, preferred_element_type=jnp.float32)
    s = jax.lax.dot_general(
        q.astype(jnp.bfloat16), k_ref[...].astype(jnp.bfloat16),
        (((1,), (1,)), ((), ())),
        preferred_element_type=jnp.float32) * (1.0 / math.sqrt(RANK))

    iota_c = jax.lax.broadcasted_iota(jnp.int32, (TS, NCLS), 1)
    iota_g = jax.lax.broadcasted_iota(jnp.int32, (TS, N_KNOWLEDGE), 1)

    p1, b1, p2, b2, p3, b3 = _fold_top3(s)
    vals = []
    idxs = []
    cnt = jnp.zeros((TS, NCLS), jnp.int32)
    for j in range(K_KNOW):
        m = jnp.max(p1, axis=1, keepdims=True)
        gcand = jnp.where(p1 == m, b1 * NCLS + iota_c, BIGIDX)
        a = jnp.min(gcand, axis=1, keepdims=True)
        vals.append(m)
        idxs.append(a)
        g = jnp.bitwise_and(a, NCLS - 1)
        hit = iota_c == g
        cnt = cnt + hit.astype(jnp.int32)
        p1 = jnp.where(hit, p2, p1)
        b1 = jnp.where(hit, b2, b1)
        p2 = jnp.where(hit, p3, p2)
        b2 = jnp.where(hit, b3, b2)
        p3 = jnp.where(hit, NEG, p3)

    v8 = jnp.concatenate(vals, axis=1)
    i8 = jnp.concatenate(idxs, axis=1)
    e8 = jnp.exp(v8 - v8[:, 0:1])
    w8 = e8 / jnp.sum(e8, axis=1, keepdims=True)
    idx_ref[0] = i8
    w_ref[0] = w8

    bad = jnp.any(cnt >= 3)

    @pl.when(bad)
    def _slow_exact():
        ss = s
        nvals = []
        nidxs = []
        for _ in range(K_KNOW):
            nm = jnp.max(ss, axis=1, keepdims=True)
            ncand = jnp.where(ss == nm, iota_g, BIGIDX)
            na = jnp.min(ncand, axis=1, keepdims=True)
            nvals.append(nm)
            nidxs.append(na)
            ss = jnp.where(iota_g == na, NEG, ss)
        nv8 = jnp.concatenate(nvals, axis=1)
        ni8 = jnp.concatenate(nidxs, axis=1)
        ne8 = jnp.exp(nv8 - nv8[:, 0:1])
        nw8 = ne8 / jnp.sum(ne8, axis=1, keepdims=True)
        idx_ref[0] = ni8
        w_ref[0] = nw8


def _make_sc_combine(tpw, nbatch):
    def _sc_combine(idx_hbm, w_hbm, v_hbm, out_hbm,
                    idx_v, w_v, rows_v, outb_v, rsem, osem):
        wid = lax.axis_index("s") * 2 + lax.axis_index("c")
        base = wid * tpw
        pltpu.sync_copy(idx_hbm.at[wid], idx_v)
        pltpu.sync_copy(w_hbm.at[wid], w_v)

        pltpu.async_copy(v_hbm.at[idx_v.at[0]], rows_v.at[0], rsem.at[0])
        pltpu.async_copy(v_hbm.at[idx_v.at[1]], rows_v.at[1], rsem.at[1])

        def batch_pair(gg, carry):
            for par in range(2):
                g = gg * 2 + par
                pltpu.make_async_copy(
                    v_hbm.at[idx_v.at[g]], rows_v.at[par], rsem.at[par]).wait()
                @pl.when(gg > 0)
                def _():
                    pltpu.make_async_copy(
                        outb_v.at[par],
                        out_hbm.at[pl.ds(base, GB)],
                        osem.at[par]).wait()

                wpos0 = g * (GB * K_KNOW)
                w_chunks = [w_v[pl.ds(wpos0, 16)], w_v[pl.ds(wpos0 + 16, 16)]]
                ws = []
                for t in range(GB):
                    wst = []
                    for j in range(K_KNOW):
                        p = t * K_KNOW + j
                        lane = jnp.full((16,), p % 16, jnp.int32)
                        wst.append(w_chunks[p // 16].at[lane]
                                   .get(mode="promise_in_bounds"))
                    ws.append(wst)

                def cbody(c, _):
                    sl = pl.ds(c * 16, 16)
                    for t in range(GB):
                        a = rows_v[par, t * K_KNOW + 0, sl] * ws[t][0]
                        for j in range(1, K_KNOW):
                            a = a + rows_v[par, t * K_KNOW + j, sl] * ws[t][j]
                        outb_v[par, t, sl] = a
                    return 0

                jax.lax.fori_loop(0, D_MODEL // 16, cbody, 0)

                pltpu.async_copy(
                    outb_v.at[par],
                    out_hbm.at[pl.ds(base + g * GB, GB)],
                    osem.at[par])
                @pl.when(gg < nbatch // 2 - 1)
                def _():
                    pltpu.async_copy(
                        v_hbm.at[idx_v.at[g + 2]], rows_v.at[par], rsem.at[par])
            return carry

        jax.lax.fori_loop(0, nbatch // 2, batch_pair, 0)

        for par in range(2):
            pltpu.make_async_copy(
                outb_v.at[par], out_hbm.at[pl.ds(base, GB)], osem.at[par]).wait()

    return _sc_combine


NCH = 8
S_CH = S // NCH


def kernel(x, memory_topk_w, memory_topk_idx, compress_neurons, knowledge_K, knowledge_V):
    cn2 = compress_neurons.reshape(N_COMPRESS, D_MODEL * RANK)
    shared_flat = pl.pallas_call(
        _mix_kernel,
        grid=(16,),
        in_specs=[
            pl.BlockSpec((B, TOPK_C), lambda i: (0, 0)),
            pl.BlockSpec((B, TOPK_C), lambda i: (0, 0)),
            pl.BlockSpec((N_COMPRESS, D_MODEL * RANK // 16), lambda i: (0, i)),
        ],
        out_specs=pl.BlockSpec((B, D_MODEL * RANK // 16), lambda i: (0, i)),
        out_shape=jax.ShapeDtypeStruct((B, D_MODEL * RANK), jnp.float32),
    )(memory_topk_w, memory_topk_idx, cn2)
    shared_compress = shared_flat.reshape(B, D_MODEL, RANK)

    tok_ch = B * S_CH
    tpw = tok_ch // NW
    nbatch = tpw // GB
    mesh = plsc.VectorSubcoreMesh(core_axis_name="c", subcore_axis_name="s")
    sc_body = _make_sc_combine(tpw, nbatch)

    out_chunks, idx_chunks, w_chunks = [], [], []
    for c in range(NCH):
        xc = x[:, c * S_CH:(c + 1) * S_CH]
        topk_idx_c, weights_c = pl.pallas_call(
            _topk_kernel,
            grid=(B, S_CH // TS),
            in_specs=[
                pl.BlockSpec((1, TS, D_MODEL), lambda b, s: (b, s, 0)),
                pl.BlockSpec((1, D_MODEL, RANK), lambda b, s: (b, 0, 0)),
                pl.BlockSpec((N_KNOWLEDGE, RANK), lambda b, s: (0, 0)),
            ],
            out_specs=[
                pl.BlockSpec((1, TS, K_KNOW), lambda b, s: (b, s, 0)),
                pl.BlockSpec((1, TS, K_KNOW), lambda b, s: (b, s, 0)),
            ],
            out_shape=[
                jax.ShapeDtypeStruct((B, S_CH, K_KNOW), jnp.int32),
                jax.ShapeDtypeStruct((B, S_CH, K_KNOW), jnp.float32),
            ],
        )(xc, shared_compress, knowledge_K)

        idx_w = topk_idx_c.reshape(NW, nbatch, ROWS_PER_BATCH)
        w_w = weights_c.reshape(NW, tpw * K_KNOW)
        out_flat = pl.kernel(
            sc_body,
            mesh=mesh,
            out_type=jax.ShapeDtypeStruct((tok_ch, D_MODEL), jnp.float32),
            scratch_types=[
                pltpu.VMEM((nbatch, ROWS_PER_BATCH), jnp.int32),
                pltpu.VMEM((tpw * K_KNOW,), jnp.float32),
                pltpu.VMEM((2, ROWS_PER_BATCH, D_MODEL), jnp.float32),
                pltpu.VMEM((2, GB, D_MODEL), jnp.float32),
                pltpu.SemaphoreType.DMA((2,)),
                pltpu.SemaphoreType.DMA((2,)),
            ],
        )(idx_w, w_w, knowledge_V)
        out_chunks.append(out_flat.reshape(B, S_CH, D_MODEL))
        idx_chunks.append(topk_idx_c)
        w_chunks.append(weights_c)

    output = jnp.concatenate(out_chunks, axis=1)
    topk_idx = jnp.concatenate(idx_chunks, axis=1)
    weights = jnp.concatenate(w_chunks, axis=1)
    return (output, topk_idx, weights)
